# baseline (speedup 1.0000x reference)
import functools

import jax
import jax.numpy as jnp
from jax import lax
from jax.experimental import pallas as pl
from jax.experimental.pallas import tpu as pltpu

_E = 8
_NEG_INF = float("-inf")


def _moe_body(x_ref, gw_ref, b_ref, a_ref, out_ref, xb_ref, wcol_ref):
    e = pl.program_id(0)

    @pl.when(e == 0)
    def _routing():
        xt = x_ref[...]
        xb_ref[...] = xt.astype(jnp.bfloat16)
        logits = lax.dot_general(xt, gw_ref[...],
                                 (((1,), (1,)), ((), ())),
                                 preferred_element_type=jnp.float32)
        lane = jax.lax.broadcasted_iota(jnp.int32, logits.shape, 1)
        m1 = jnp.max(logits, axis=1, keepdims=True)
        idx0 = jnp.min(jnp.where(logits == m1, lane, _E), axis=1,
                       keepdims=True)
        logits2 = jnp.where(lane == idx0, _NEG_INF, logits)
        m2 = jnp.max(logits2, axis=1, keepdims=True)
        idx1 = jnp.min(jnp.where(logits2 == m2, lane, _E), axis=1,
                       keepdims=True)
        s = jnp.exp(m2 - m1)
        w0 = 1.0 / (1.0 + s)
        w1 = 1.0 - w0
        dense_w = (jnp.where(lane == idx0, w0, 0.0)
                   + jnp.where(lane == idx1, w1, 0.0))
        for ee in range(_E):
            wcol_ref[ee] = dense_w[:, ee:ee + 1]
        out_ref[...] = jnp.dot(dense_w, b_ref[...],
                               preferred_element_type=jnp.float32)

    y = jax.lax.dot_general(xb_ref[...], a_ref[0], (((1,), (0,)), ((), ())),
                            preferred_element_type=jnp.float32)
    out_ref[...] += wcol_ref[e] * y


@functools.partial(jax.jit, static_argnames=())
def kernel(inputs, gate_w, expert_A, expert_b):
    batch_shape = inputs.shape[:-1]
    d = inputs.shape[-1]
    x = inputs.reshape(-1, d)
    t = x.shape[0]

    out = pl.pallas_call(
        _moe_body,
        grid=(_E,),
        in_specs=[
            pl.BlockSpec((t, d), lambda e: (0, 0)),
            pl.BlockSpec((_E, d), lambda e: (0, 0)),
            pl.BlockSpec((_E, d), lambda e: (0, 0)),
            pl.BlockSpec((1, d, d), lambda e: (e, 0, 0)),
        ],
        out_specs=pl.BlockSpec((t, d), lambda e: (0, 0)),
        out_shape=jax.ShapeDtypeStruct((t, d), jnp.float32),
        scratch_shapes=[
            pltpu.VMEM((t, d), jnp.bfloat16),
            pltpu.VMEM((_E, t, 1), jnp.float32),
        ],
    )(x, gate_w, expert_b, expert_A)
    return out.reshape(*batch_shape, d)
